# Initial kernel scaffold; baseline (speedup 1.0000x reference)
import jax
import jax.numpy as jnp
from jax.experimental import pallas as pl


def kernel(nodes, edge_nodes, X, w_att_w, w_att_b, lin_W, lin_b, skip_W, beta, cls_W, cls_b):
    raise NotImplementedError("write your pallas kernel here")


# speedup vs baseline: 4.7039x; 4.7039x over previous
import functools

import jax
import jax.numpy as jnp
from jax import lax
from jax.experimental import pallas as pl
from jax.experimental.pallas import tpu as pltpu
from jax.experimental.pallas import tpu_sc as plsc

N = 50000
D = 256
B = 4096
E = 16
M = 4
HID = 256
NC = 40

NCORES = 2
NSUB = 16
NW = NCORES * NSUB
BPW = B // NW
EM = E * M
LANES = 16
DCH = D // LANES
NBUF = 2


def _sc_gather_combine(X, eidx, nidx, wvec, bvec):
    f32 = jnp.float32
    mesh = plsc.VectorSubcoreMesh(core_axis_name="c", subcore_axis_name="s")
    out_type = (
        jax.ShapeDtypeStruct((B, D), f32),
        jax.ShapeDtypeStruct((B, D), f32),
    )
    scratch = [
        pltpu.VMEM((BPW * EM,), jnp.int32),
        pltpu.VMEM((BPW,), jnp.int32),
        pltpu.VMEM((NBUF, EM, D), f32),
        pltpu.VMEM((BPW, D), f32),
        pltpu.VMEM((BPW, D), f32),
        pltpu.VMEM((LANES,), f32),
        pltpu.VMEM((LANES,), f32),
        pltpu.SemaphoreType.DMA,
        pltpu.SemaphoreType.DMA,
        pltpu.SemaphoreType.DMA,
    ]

    @functools.partial(pl.kernel, out_type=out_type, mesh=mesh,
                       scratch_types=scratch)
    def k(Xh, eidxh, nidxh, wvh, bvh, self_out, neigh_out,
          idx_v, nidx_v, rows, selfr, neigh, wv_v, bv_v, sem0, sem1, semself):
        wid = lax.axis_index("s") * NCORES + lax.axis_index("c")
        base = wid * BPW
        pltpu.sync_copy(eidxh.at[pl.ds(base * EM, BPW * EM)], idx_v)
        pltpu.sync_copy(nidxh.at[pl.ds(base, BPW)], nidx_v)
        pltpu.sync_copy(wvh, wv_v)
        pltpu.sync_copy(bvh, bv_v)
        selfcp = pltpu.async_copy(Xh.at[nidx_v], selfr, semself)
        wv = wv_v[...]
        bv = bv_v[...]
        sems = [sem0, sem1]

        for s_ in range(NBUF):
            pltpu.async_copy(Xh.at[idx_v.at[pl.ds(s_ * EM, EM)]],
                             rows.at[s_], sems[s_])

        @pl.loop(0, BPW, step=NBUF)
        def _gloop(g):
            for s_ in range(NBUF):
                b = g + s_
                slot = rows.at[s_]
                pltpu.make_async_copy(
                    Xh.at[idx_v.at[pl.ds(b * EM, EM)]], slot,
                    sems[s_]).wait()

                def ebody(e, accs):
                    va = jnp.zeros((LANES,), f32)
                    msgs = []
                    for dc in range(DCH):
                        sl = pl.ds(dc * LANES, LANES)
                        f0 = slot[4 * e + 0, sl]
                        f1 = slot[4 * e + 1, sl]
                        f2 = slot[4 * e + 2, sl]
                        f3 = slot[4 * e + 3, sl]
                        ss = (f0 + f1) + (f2 + f3)
                        qq = (f0 * f0 + f1 * f1) + (f2 * f2 + f3 * f3)
                        mu = ss * 0.25
                        va = va + (qq * 0.25 - mu * mu)
                        msgs.append(f0 * f1 * f2)
                    lanes = lax.iota(jnp.int32, LANES)
                    dnums = lax.GatherDimensionNumbers(
                        offset_dims=(), collapsed_slice_dims=(0,),
                        start_index_map=(0,))
                    for sh in (8, 4, 2, 1):
                        perm = (lanes + sh) & (LANES - 1)
                        va = va + lax.gather(
                            va, perm[:, None], dnums, slice_sizes=(1,),
                            mode=lax.GatherScatterMode.PROMISE_IN_BOUNDS)
                    z = va * wv + bv
                    att = 1.0 / (1.0 + jnp.exp(-z))
                    return tuple(accs[dc] + att * msgs[dc]
                                 for dc in range(DCH))

                init = tuple(jnp.zeros((LANES,), f32) for _ in range(DCH))
                accs = lax.fori_loop(0, E, ebody, init)
                for dc in range(DCH):
                    neigh[b, pl.ds(dc * LANES, LANES)] = accs[dc]

                nb = b + NBUF

                @pl.when(nb < BPW)
                def _():
                    pltpu.async_copy(Xh.at[idx_v.at[pl.ds(nb * EM, EM)]],
                                     rows.at[s_], sems[s_])

        selfcp.wait()
        pltpu.sync_copy(neigh, neigh_out.at[pl.ds(base, BPW)])
        pltpu.sync_copy(selfr, self_out.at[pl.ds(base, BPW)])

    return k(X, eidx, nidx, wvec, bvec)


def _tc_dense(self_feat, neigh, w1, w2, lin_b2, skip_W, beta2, cls_Wp, cls_bp):
    f32 = jnp.float32
    BT = 512
    grid = (B // BT,)

    def body(beta_ref, s_ref, n_ref, w1_ref, w2_ref, lb_ref, sw_ref,
             cw_ref, cb_ref, out_ref):
        x_s = s_ref[...]
        x_n = n_ref[...]
        h = (jnp.dot(x_s, w1_ref[...], preferred_element_type=f32)
             + jnp.dot(x_n, w2_ref[...], preferred_element_type=f32)
             + lb_ref[...])
        out = jnp.maximum(h, 0.0)
        bt = beta_ref[0, 0]
        enc = ((1.0 - bt) * out
               + bt * jnp.dot(x_s, sw_ref[...], preferred_element_type=f32))
        logits = jnp.dot(enc, cw_ref[...], preferred_element_type=f32) + cb_ref[...]
        mx = jnp.max(logits, axis=1, keepdims=True)
        ex = jnp.exp(logits - mx)
        lse = jnp.log(jnp.sum(ex, axis=1, keepdims=True)) + mx
        out_ref[...] = logits - lse

    return pl.pallas_call(
        body,
        grid=grid,
        in_specs=[
            pl.BlockSpec(memory_space=pltpu.SMEM),
            pl.BlockSpec((BT, D), lambda i: (i, 0)),
            pl.BlockSpec((BT, D), lambda i: (i, 0)),
            pl.BlockSpec((D, HID), lambda i: (0, 0)),
            pl.BlockSpec((D, HID), lambda i: (0, 0)),
            pl.BlockSpec((1, HID), lambda i: (0, 0)),
            pl.BlockSpec((D, HID), lambda i: (0, 0)),
            pl.BlockSpec((HID, 128), lambda i: (0, 0)),
            pl.BlockSpec((1, 128), lambda i: (0, 0)),
        ],
        out_specs=pl.BlockSpec((BT, 128), lambda i: (i, 0)),
        out_shape=jax.ShapeDtypeStruct((B, 128), f32),
    )(beta2, self_feat, neigh, w1, w2, lin_b2, skip_W, cls_Wp, cls_bp)


def kernel(nodes, edge_nodes, X, w_att_w, w_att_b, lin_W, lin_b, skip_W,
           beta, cls_W, cls_b):
    f32 = jnp.float32
    nidx = nodes.astype(jnp.int32)
    eidx = edge_nodes.astype(jnp.int32).reshape(-1)
    wvec = jnp.full((LANES,), w_att_w[0, 0] / D, f32)
    bvec = jnp.full((LANES,), w_att_b[0], f32)

    self_feat, neigh = _sc_gather_combine(X, eidx, nidx, wvec, bvec)

    w1 = lin_W[:D]
    w2 = lin_W[D:]
    lin_b2 = lin_b.reshape(1, HID)
    beta2 = jnp.reshape(beta, (1, 1)).astype(f32)
    cls_Wp = jnp.zeros((HID, 128), f32).at[:, :NC].set(cls_W)
    cls_bp = jnp.full((1, 128), -1e30, f32).at[0, :NC].set(cls_b)

    logp = _tc_dense(self_feat, neigh, w1, w2, lin_b2, skip_W, beta2,
                     cls_Wp, cls_bp)
    return logp[:, :NC]


# speedup vs baseline: 6.9064x; 1.4682x over previous
import functools

import jax
import jax.numpy as jnp
from jax import lax
from jax.experimental import pallas as pl
from jax.experimental.pallas import tpu as pltpu
from jax.experimental.pallas import tpu_sc as plsc

N = 50000
D = 256
B = 4096
E = 16
M = 4
HID = 256
NC = 40

NCORES = 2
NSUB = 16
NW = NCORES * NSUB
BPW = B // NW
EM = E * M
LANES = 16
DCH = D // LANES
NBUF = 2


def _sc_gather_combine(X, eidx, nidx, wvec, bvec):
    f32 = jnp.float32
    mesh = plsc.VectorSubcoreMesh(core_axis_name="c", subcore_axis_name="s")
    out_type = (
        jax.ShapeDtypeStruct((B, D), f32),
        jax.ShapeDtypeStruct((B, D), f32),
    )
    scratch = [
        pltpu.VMEM((BPW * EM,), jnp.int32),
        pltpu.VMEM((BPW,), jnp.int32),
        pltpu.VMEM((NBUF, EM, D), f32),
        pltpu.VMEM((BPW, D), f32),
        pltpu.VMEM((BPW, D), f32),
        pltpu.VMEM((LANES,), f32),
        pltpu.VMEM((LANES,), f32),
        pltpu.SemaphoreType.DMA,
        pltpu.SemaphoreType.DMA,
        pltpu.SemaphoreType.DMA,
    ]

    @functools.partial(pl.kernel, out_type=out_type, mesh=mesh,
                       scratch_types=scratch)
    def k(Xh, eidxh, nidxh, wvh, bvh, self_out, neigh_out,
          idx_v, nidx_v, rows, selfr, neigh, wv_v, bv_v, sem0, sem1, semself):
        wid = lax.axis_index("s") * NCORES + lax.axis_index("c")
        base = wid * BPW
        pltpu.sync_copy(eidxh.at[pl.ds(base * EM, BPW * EM)], idx_v)
        pltpu.sync_copy(nidxh.at[pl.ds(base, BPW)], nidx_v)
        pltpu.sync_copy(wvh, wv_v)
        pltpu.sync_copy(bvh, bv_v)
        selfcp = pltpu.async_copy(Xh.at[nidx_v], selfr, semself)
        wv = wv_v[...]
        bv = bv_v[...]
        sems = [sem0, sem1]

        for s_ in range(NBUF):
            pltpu.async_copy(Xh.at[idx_v.at[pl.ds(s_ * EM, EM)]],
                             rows.at[s_], sems[s_])

        @pl.loop(0, BPW, step=NBUF)
        def _gloop(g):
            for s_ in range(NBUF):
                b = g + s_
                slot = rows.at[s_]
                pltpu.make_async_copy(
                    Xh.at[idx_v.at[pl.ds(b * EM, EM)]], slot,
                    sems[s_]).wait()

                zero = jnp.zeros((LANES,), f32)
                for dc in range(DCH):
                    neigh[b, pl.ds(dc * LANES, LANES)] = zero

                def ebody(e, carry):
                    va = jnp.zeros((LANES,), f32)
                    msgs = []
                    for dc in range(DCH):
                        sl = pl.ds(dc * LANES, LANES)
                        f0 = slot[4 * e + 0, sl]
                        f1 = slot[4 * e + 1, sl]
                        f2 = slot[4 * e + 2, sl]
                        f3 = slot[4 * e + 3, sl]
                        ss = (f0 + f1) + (f2 + f3)
                        qq = (f0 * f0 + f1 * f1) + (f2 * f2 + f3 * f3)
                        mu = ss * 0.25
                        va = va + (qq * 0.25 - mu * mu)
                        msgs.append(f0 * f1 * f2)
                    lanes = lax.iota(jnp.int32, LANES)
                    dnums = lax.GatherDimensionNumbers(
                        offset_dims=(), collapsed_slice_dims=(0,),
                        start_index_map=(0,))
                    for sh in (8, 4, 2, 1):
                        perm = (lanes + sh) & (LANES - 1)
                        va = va + lax.gather(
                            va, perm[:, None], dnums, slice_sizes=(1,),
                            mode=lax.GatherScatterMode.PROMISE_IN_BOUNDS)
                    z = va * wv + bv
                    att = 1.0 / (1.0 + jnp.exp(-z))
                    for dc in range(DCH):
                        plsc.addupdate(neigh.at[b, pl.ds(dc * LANES, LANES)],
                                       att * msgs[dc])
                    return carry

                lax.fori_loop(0, E, ebody, 0)

                nb = b + NBUF

                @pl.when(nb < BPW)
                def _():
                    pltpu.async_copy(Xh.at[idx_v.at[pl.ds(nb * EM, EM)]],
                                     rows.at[s_], sems[s_])

        selfcp.wait()
        pltpu.sync_copy(neigh, neigh_out.at[pl.ds(base, BPW)])
        pltpu.sync_copy(selfr, self_out.at[pl.ds(base, BPW)])

    return k(X, eidx, nidx, wvec, bvec)


def _tc_dense(self_feat, neigh, w1, w2, lin_b2, skip_W, beta2, cls_Wp, cls_bp):
    f32 = jnp.float32
    BT = 512
    grid = (B // BT,)

    def body(beta_ref, s_ref, n_ref, w1_ref, w2_ref, lb_ref, sw_ref,
             cw_ref, cb_ref, out_ref):
        x_s = s_ref[...]
        x_n = n_ref[...]
        h = (jnp.dot(x_s, w1_ref[...], preferred_element_type=f32)
             + jnp.dot(x_n, w2_ref[...], preferred_element_type=f32)
             + lb_ref[...])
        out = jnp.maximum(h, 0.0)
        bt = beta_ref[0, 0]
        enc = ((1.0 - bt) * out
               + bt * jnp.dot(x_s, sw_ref[...], preferred_element_type=f32))
        logits = jnp.dot(enc, cw_ref[...], preferred_element_type=f32) + cb_ref[...]
        mx = jnp.max(logits, axis=1, keepdims=True)
        ex = jnp.exp(logits - mx)
        lse = jnp.log(jnp.sum(ex, axis=1, keepdims=True)) + mx
        out_ref[...] = logits - lse

    return pl.pallas_call(
        body,
        grid=grid,
        in_specs=[
            pl.BlockSpec(memory_space=pltpu.SMEM),
            pl.BlockSpec((BT, D), lambda i: (i, 0)),
            pl.BlockSpec((BT, D), lambda i: (i, 0)),
            pl.BlockSpec((D, HID), lambda i: (0, 0)),
            pl.BlockSpec((D, HID), lambda i: (0, 0)),
            pl.BlockSpec((1, HID), lambda i: (0, 0)),
            pl.BlockSpec((D, HID), lambda i: (0, 0)),
            pl.BlockSpec((HID, 128), lambda i: (0, 0)),
            pl.BlockSpec((1, 128), lambda i: (0, 0)),
        ],
        out_specs=pl.BlockSpec((BT, 128), lambda i: (i, 0)),
        out_shape=jax.ShapeDtypeStruct((B, 128), f32),
    )(beta2, self_feat, neigh, w1, w2, lin_b2, skip_W, cls_Wp, cls_bp)


def kernel(nodes, edge_nodes, X, w_att_w, w_att_b, lin_W, lin_b, skip_W,
           beta, cls_W, cls_b):
    f32 = jnp.float32
    nidx = nodes.astype(jnp.int32)
    eidx = edge_nodes.astype(jnp.int32).reshape(-1)
    wvec = jnp.full((LANES,), w_att_w[0, 0] / D, f32)
    bvec = jnp.full((LANES,), w_att_b[0], f32)

    self_feat, neigh = _sc_gather_combine(X, eidx, nidx, wvec, bvec)

    w1 = lin_W[:D]
    w2 = lin_W[D:]
    lin_b2 = lin_b.reshape(1, HID)
    beta2 = jnp.reshape(beta, (1, 1)).astype(f32)
    cls_Wp = jnp.zeros((HID, 128), f32).at[:, :NC].set(cls_W)
    cls_bp = jnp.full((1, 128), -1e30, f32).at[0, :NC].set(cls_b)

    logp = _tc_dense(self_feat, neigh, w1, w2, lin_b2, skip_W, beta2,
                     cls_Wp, cls_bp)
    return logp[:, :NC]
